# baseline (speedup 1.0000x reference)
import jax
import jax.numpy as jnp
import numpy as np
from jax.experimental import pallas as pl
from jax.experimental.pallas import tpu as pltpu

_K = 64
_M = 7
_N = 4096
_EPS = 1e-7
_CLIP_HI = np.float32(1.0) - np.float32(1e-7)
_LANE = 128
_CHUNKS_PER_HALF = 16
_HALF = _LANE * _CHUNKS_PER_HALF


def _pm_kernel(inp_ref, theta_ref, cw_ref, out_ref):
    rows = inp_ref[0]
    theta = theta_ref[...]

    hom_f = rows[0:1, :]
    hom_i = hom_f.astype(jnp.int32)
    cnt = jnp.sum((rows != 0.0).astype(jnp.int32), axis=0, keepdims=True)
    valid = (hom_i <= 1) & (cnt != 0)
    iota = jax.lax.broadcasted_iota(jnp.int32, (1, _N), 1)
    first_bad = jnp.min(jnp.where(valid, _N, iota))
    prefix = (iota < first_bad).astype(jnp.float32)
    c0 = cw_ref[0, 0]
    c1 = cw_ref[0, 1]
    hc = jnp.clip(hom_i, 0, 1)
    w = jnp.where(hc >= 1, c1, c0)
    coef = w * prefix

    u = []
    for m in range(_M):
        u.append(rows[1 + m:2 + m, :] + theta[:, m:m + 1])
    sq = u[0] * u[0]
    for m in range(1, _M):
        sq = sq + u[m] * u[m]
    d = jnp.sqrt(sq + 1.0) + 1.0

    x = [u[m] / d for m in range(_M)]
    nx2 = x[0] * x[0]
    for m in range(1, _M):
        nx2 = nx2 + x[m] * x[m]
    n0 = jnp.sqrt(nx2)
    nc = jnp.clip(n0, np.float32(1e-7), _CLIP_HI)
    at = (jnp.log1p(nc) - jnp.log1p(-nc)) * 0.5

    cols = []
    for m in range(_M):
        c_m = coef * ((at * x[m]) / nc)
        s_m = None
        for h in range(2):
            base = h * _HALF
            part = c_m[:, base:base + _LANE]
            for c in range(1, _CHUNKS_PER_HALF):
                lo = base + c * _LANE
                part = part + c_m[:, lo:lo + _LANE]
            half_sum = jnp.sum(part, axis=1, keepdims=True)
            s_m = half_sum if s_m is None else s_m + half_sum
        cols.append(s_m)
    S = jnp.concatenate(cols, axis=1)

    rows_c = [S[0:1, :]]
    for k in range(1, _K):
        rows_c.append(rows_c[-1] + S[k:k + 1, :])
    cumS = jnp.concatenate(rows_c, axis=0)

    nsq = cumS[:, 0:1] * cumS[:, 0:1]
    for m in range(1, _M):
        nsq = nsq + cumS[:, m:m + 1] * cumS[:, m:m + 1]
    nrm = jnp.maximum(jnp.sqrt(nsq), np.float32(_EPS))
    th = jnp.tanh(nrm)
    e = [(th * cumS[:, m:m + 1]) / nrm for m in range(_M)]
    esq = e[0] * e[0]
    for m in range(1, _M):
        esq = esq + e[m] * e[m]
    den = (1.0 - esq) + np.float32(_EPS)
    y = [(2.0 * e[m]) / den for m in range(_M)]
    out_ref[...] = jnp.concatenate(y, axis=1)[None]


def kernel(input, theta, class_w):
    b = input.shape[0]
    inp_t = jnp.transpose(input, (0, 2, 1))
    cw2 = class_w.reshape(1, 2).astype(jnp.float32)
    y = pl.pallas_call(
        _pm_kernel,
        grid=(b,),
        in_specs=[
            pl.BlockSpec((1, _M + 1, _N), lambda i: (i, 0, 0)),
            pl.BlockSpec((_K, _M), lambda i: (0, 0)),
            pl.BlockSpec((1, 2), lambda i: (0, 0)),
        ],
        out_specs=pl.BlockSpec((1, _K, _M), lambda i: (i, 0, 0)),
        out_shape=jax.ShapeDtypeStruct((b, _K, _M), jnp.float32),
        compiler_params=pltpu.CompilerParams(
            dimension_semantics=("parallel",)),
    )(inp_t, theta, cw2)
    return y.reshape(b, _K * _M)


# speedup vs baseline: 1.0599x; 1.0599x over previous
import jax
import jax.numpy as jnp
import numpy as np
from jax.experimental import pallas as pl
from jax.experimental.pallas import tpu as pltpu

_K = 64
_M = 7
_N = 4096
_EPS = 1e-7
_CLIP_HI = np.float32(1.0) - np.float32(1e-7)
_LANE = 128
_CHUNKS_PER_HALF = 16
_HALF = _LANE * _CHUNKS_PER_HALF


def _pm_body(rows, theta, cw_ref):
    hom_f = rows[0:1, :]
    hom_i = hom_f.astype(jnp.int32)
    cnt = jnp.sum((rows != 0.0).astype(jnp.int32), axis=0, keepdims=True)
    valid = (hom_i <= 1) & (cnt != 0)
    iota = jax.lax.broadcasted_iota(jnp.int32, (1, _N), 1)
    first_bad = jnp.min(jnp.where(valid, _N, iota))
    prefix = (iota < first_bad).astype(jnp.float32)
    c0 = cw_ref[0, 0]
    c1 = cw_ref[0, 1]
    hc = jnp.clip(hom_i, 0, 1)
    w = jnp.where(hc >= 1, c1, c0)
    coef = w * prefix

    u = []
    for m in range(_M):
        u.append(rows[1 + m:2 + m, :] + theta[:, m:m + 1])
    sq = u[0] * u[0]
    for m in range(1, _M):
        sq = sq + u[m] * u[m]
    d = jnp.sqrt(sq + 1.0) + 1.0

    x = [u[m] / d for m in range(_M)]
    nx2 = x[0] * x[0]
    for m in range(1, _M):
        nx2 = nx2 + x[m] * x[m]
    n0 = jnp.sqrt(nx2)
    nc = jnp.clip(n0, np.float32(1e-7), _CLIP_HI)
    at = (jnp.log1p(nc) - jnp.log1p(-nc)) * 0.5

    cols = []
    for m in range(_M):
        c_m = coef * ((at * x[m]) / nc)
        s_m = None
        for h in range(2):
            base = h * _HALF
            part = c_m[:, base:base + _LANE]
            for c in range(1, _CHUNKS_PER_HALF):
                lo = base + c * _LANE
                part = part + c_m[:, lo:lo + _LANE]
            half_sum = jnp.sum(part, axis=1, keepdims=True)
            s_m = half_sum if s_m is None else s_m + half_sum
        cols.append(s_m)
    return jnp.concatenate(cols, axis=1)


def _pm_kernel(inp_ref, theta_t_ref, cw_ref, out_ref):
    theta = jnp.transpose(theta_t_ref[...], (1, 0))
    for j in range(_BB):
        rows = inp_ref[j]
        out_ref[j] = _pm_body(rows, theta, cw_ref)


_BB = 2


def kernel(input, theta, class_w):
    b = input.shape[0]
    inp_t = jnp.transpose(input, (0, 2, 1))
    theta_t = jnp.transpose(theta, (1, 0))
    cw2 = class_w.reshape(1, 2).astype(jnp.float32)
    s = pl.pallas_call(
        _pm_kernel,
        grid=(b // _BB,),
        in_specs=[
            pl.BlockSpec((_BB, _M + 1, _N), lambda i: (i, 0, 0)),
            pl.BlockSpec((_M, _K), lambda i: (0, 0)),
            pl.BlockSpec((1, 2), lambda i: (0, 0)),
        ],
        out_specs=pl.BlockSpec((_BB, _K, _M), lambda i: (i, 0, 0)),
        out_shape=jax.ShapeDtypeStruct((b, _K, _M), jnp.float32),
        compiler_params=pltpu.CompilerParams(
            dimension_semantics=("parallel",)),
    )(inp_t, theta_t, cw2)

    cumS = jnp.cumsum(s, axis=1)
    n = jnp.sqrt(jnp.sum(cumS * cumS, axis=-1, keepdims=True))
    n = jnp.maximum(n, _EPS)
    e = jnp.tanh(n) * cumS / n
    sq = jnp.sum(e * e, axis=-1, keepdims=True)
    y = 2.0 * e / (1.0 - sq + _EPS)
    return y.reshape(b, _K * _M)


# speedup vs baseline: 1.0600x; 1.0001x over previous
import jax
import jax.numpy as jnp
import numpy as np
from jax.experimental import pallas as pl
from jax.experimental.pallas import tpu as pltpu

_K = 64
_M = 7
_N = 4096
_EPS = 1e-7
_CLIP_HI = np.float32(1.0) - np.float32(1e-7)
_LANE = 128
_CHUNKS_PER_HALF = 16
_HALF = _LANE * _CHUNKS_PER_HALF


def _pm_body(rows, theta, cw_ref):
    hom_f = rows[0:1, :]
    hom_i = hom_f.astype(jnp.int32)
    cnt = jnp.sum((rows != 0.0).astype(jnp.int32), axis=0, keepdims=True)
    valid = (hom_i <= 1) & (cnt != 0)
    iota = jax.lax.broadcasted_iota(jnp.int32, (1, _N), 1)
    first_bad = jnp.min(jnp.where(valid, _N, iota))
    prefix = (iota < first_bad).astype(jnp.float32)
    c0 = cw_ref[0, 0]
    c1 = cw_ref[0, 1]
    hc = jnp.clip(hom_i, 0, 1)
    w = jnp.where(hc >= 1, c1, c0)
    coef = w * prefix

    u = []
    for m in range(_M):
        u.append(rows[1 + m:2 + m, :] + theta[:, m:m + 1])
    sq = u[0] * u[0]
    for m in range(1, _M):
        sq = sq + u[m] * u[m]
    d = jnp.sqrt(sq + 1.0) + 1.0

    x = [u[m] / d for m in range(_M)]
    nx2 = x[0] * x[0]
    for m in range(1, _M):
        nx2 = nx2 + x[m] * x[m]
    n0 = jnp.sqrt(nx2)
    nc = jnp.clip(n0, np.float32(1e-7), _CLIP_HI)
    at = (jnp.log1p(nc) - jnp.log1p(-nc)) * 0.5

    cols = []
    for m in range(_M):
        c_m = coef * ((at * x[m]) / nc)
        s_m = None
        for h in range(2):
            base = h * _HALF
            part = c_m[:, base:base + _LANE]
            for c in range(1, _CHUNKS_PER_HALF):
                lo = base + c * _LANE
                part = part + c_m[:, lo:lo + _LANE]
            half_sum = jnp.sum(part, axis=1, keepdims=True)
            s_m = half_sum if s_m is None else s_m + half_sum
        cols.append(s_m)
    return jnp.concatenate(cols, axis=1)


def _pm_kernel(inp_ref, theta_t_ref, cw_ref, out_ref):
    theta = jnp.transpose(theta_t_ref[...], (1, 0))
    for j in range(_BB):
        rows = inp_ref[j]
        out_ref[j] = _pm_body(rows, theta, cw_ref)


_BB = 4


def kernel(input, theta, class_w):
    b = input.shape[0]
    inp_t = jnp.transpose(input, (0, 2, 1))
    theta_t = jnp.transpose(theta, (1, 0))
    cw2 = class_w.reshape(1, 2).astype(jnp.float32)
    s = pl.pallas_call(
        _pm_kernel,
        grid=(b // _BB,),
        in_specs=[
            pl.BlockSpec((_BB, _M + 1, _N), lambda i: (i, 0, 0)),
            pl.BlockSpec((_M, _K), lambda i: (0, 0)),
            pl.BlockSpec((1, 2), lambda i: (0, 0)),
        ],
        out_specs=pl.BlockSpec((_BB, _K, _M), lambda i: (i, 0, 0)),
        out_shape=jax.ShapeDtypeStruct((b, _K, _M), jnp.float32),
        compiler_params=pltpu.CompilerParams(
            dimension_semantics=("parallel",)),
    )(inp_t, theta_t, cw2)

    cumS = jnp.cumsum(s, axis=1)
    n = jnp.sqrt(jnp.sum(cumS * cumS, axis=-1, keepdims=True))
    n = jnp.maximum(n, _EPS)
    e = jnp.tanh(n) * cumS / n
    sq = jnp.sum(e * e, axis=-1, keepdims=True)
    y = 2.0 * e / (1.0 - sq + _EPS)
    return y.reshape(b, _K * _M)
